# baseline (speedup 1.0000x reference)
import functools

import jax
import jax.numpy as jnp
import numpy as np
from jax.experimental import pallas as pl
from jax.experimental.pallas import tpu as pltpu

_VMEM_LIMIT = 48 * 1024 * 1024


def _ru(x, m):
    return (x + m - 1) // m * m


def _fire_apply(a, mask, wsq, bsq, w1, b1, w3, b3, s_scr, *, Wp, M, p0):
    sq = wsq.shape[1]
    s = jnp.dot(a, wsq, preferred_element_type=jnp.float32)
    s = jnp.maximum(s + bsq, 0.0)
    s = (s * mask).astype(jnp.bfloat16)
    s_scr[:p0] = jnp.zeros((p0, sq), jnp.bfloat16)
    s_scr[p0:p0 + M] = s
    s_scr[p0 + M:] = jnp.zeros((s_scr.shape[0] - p0 - M, sq), jnp.bfloat16)
    acc1 = jnp.dot(s, w1, preferred_element_type=jnp.float32) + b1
    acc3 = jnp.dot(s, w3[4 * sq:5 * sq, :], preferred_element_type=jnp.float32)
    for t in range(9):
        if t == 4:
            continue
        off = p0 + (t // 3 - 1) * Wp + (t % 3 - 1)
        acc3 = acc3 + jnp.dot(s_scr[off:off + M, :], w3[t * sq:(t + 1) * sq, :],
                              preferred_element_type=jnp.float32)
    acc3 = acc3 + b3
    return jnp.concatenate([jnp.maximum(acc1, 0.0).astype(jnp.bfloat16),
                            jnp.maximum(acc3, 0.0).astype(jnp.bfloat16)], axis=-1)


def _fire_pair_kernel(x_ref, mask_ref, wsq0, bsq0, w10, b10, w30, b30,
                      wsq1, bsq1, w11, b11, w31, b31, o_ref, s_scr,
                      *, Wp, M, p0):
    mask = mask_ref[...]
    y0 = _fire_apply(x_ref[...], mask, wsq0[...], bsq0[...], w10[...], b10[...],
                     w30[...], b30[...], s_scr, Wp=Wp, M=M, p0=p0)
    o_ref[...] = _fire_apply(y0, mask, wsq1[...], bsq1[...], w11[...], b11[...],
                             w31[...], b31[...], s_scr, Wp=Wp, M=M, p0=p0)


def _fire_pair(xf, mask, f0, f1, *, H, W, G):
    Wp = W + 2
    Mp = _ru((H + 2) * Wp, 16)
    R, Cin = xf.shape
    B = R // Mp
    M = G * Mp
    p0 = _ru(Wp + 1, 16)
    sq = f0["sq_w"].shape[1]
    E = f1["e1_w"].shape[1] + f1["e3_w"].shape[1]
    wspec = lambda a: pl.BlockSpec(a.shape, lambda g: (0, 0))
    return pl.pallas_call(
        functools.partial(_fire_pair_kernel, Wp=Wp, M=M, p0=p0),
        out_shape=jax.ShapeDtypeStruct((R, E), jnp.bfloat16),
        grid=(B // G,),
        in_specs=[
            pl.BlockSpec((M, Cin), lambda g: (g, 0)),
            pl.BlockSpec((M, 1), lambda g: (g, 0)),
            wspec(f0["sq_w"]), wspec(f0["sq_b"]), wspec(f0["e1_w"]),
            wspec(f0["e1_b"]), wspec(f0["e3_w"]), wspec(f0["e3_b"]),
            wspec(f1["sq_w"]), wspec(f1["sq_b"]), wspec(f1["e1_w"]),
            wspec(f1["e1_b"]), wspec(f1["e3_w"]), wspec(f1["e3_b"]),
        ],
        out_specs=pl.BlockSpec((M, E), lambda g: (g, 0)),
        scratch_shapes=[pltpu.VMEM((p0 + M + p0, sq), jnp.bfloat16)],
        compiler_params=pltpu.CompilerParams(
            dimension_semantics=("parallel",),
            vmem_limit_bytes=_VMEM_LIMIT,
        ),
    )(xf, mask,
      f0["sq_w"], f0["sq_b"], f0["e1_w"], f0["e1_b"], f0["e3_w"], f0["e3_b"],
      f1["sq_w"], f1["sq_b"], f1["e1_w"], f1["e1_b"], f1["e3_w"], f1["e3_b"])


def _tail_kernel(x_ref, mask_ref, p_ref,
                 wsq4, bsq4, w14, b14, w34, b34,
                 wsq5, bsq5, w15, b15, w35, b35,
                 wsq6, bsq6, w16, b16, w36, b36,
                 wsq7, bsq7, w17, b17, w37, b37,
                 cw_ref, cb_ref, o_ref, s48, s64, *, Wp, M, p0):
    mask = mask_ref[...]
    y = _fire_apply(x_ref[...], mask, wsq4[...], bsq4[...], w14[...], b14[...],
                    w34[...], b34[...], s48, Wp=Wp, M=M, p0=p0)
    y = _fire_apply(y, mask, wsq5[...], bsq5[...], w15[...], b15[...],
                    w35[...], b35[...], s48, Wp=Wp, M=M, p0=p0)
    y = _fire_apply(y, mask, wsq6[...], bsq6[...], w16[...], b16[...],
                    w36[...], b36[...], s64, Wp=Wp, M=M, p0=p0)
    y = _fire_apply(y, mask, wsq7[...], bsq7[...], w17[...], b17[...],
                    w37[...], b37[...], s64, Wp=Wp, M=M, p0=p0)
    acc = jnp.dot(y, cw_ref[...], preferred_element_type=jnp.float32)
    acc = jnp.maximum(acc + cb_ref[...], 0.0)
    o_ref[0] = jnp.dot(p_ref[...], acc, preferred_element_type=jnp.float32)


def _tail(xf, mask, fires, clf_w, clf_b, *, H, W, G):
    Wp = W + 2
    Mp = _ru((H + 2) * Wp, 16)
    R, Cin = xf.shape
    B = R // Mp
    M = G * Mp
    p0 = _ru(Wp + 1, 16)
    Np = clf_w.shape[1]
    mvec = np.zeros((Mp,), np.float32)
    mm = np.zeros((H + 2, W + 2), np.float32)
    mm[1:H + 1, 1:W + 1] = 1.0 / float(H * W)
    mvec[:(H + 2) * Wp] = mm.reshape(-1)
    P = np.zeros((G, M), np.float32)
    for g in range(G):
        P[g, g * Mp:(g + 1) * Mp] = mvec
    P = jnp.asarray(P)
    wspec = lambda a: pl.BlockSpec(a.shape, lambda g: (0, 0))
    args = []
    for f in fires:
        args += [f["sq_w"], f["sq_b"], f["e1_w"], f["e1_b"], f["e3_w"], f["e3_b"]]
    out = pl.pallas_call(
        functools.partial(_tail_kernel, Wp=Wp, M=M, p0=p0),
        out_shape=jax.ShapeDtypeStruct((B // G, G, Np), jnp.float32),
        grid=(B // G,),
        in_specs=[
            pl.BlockSpec((M, Cin), lambda g: (g, 0)),
            pl.BlockSpec((M, 1), lambda g: (g, 0)),
            pl.BlockSpec((G, M), lambda g: (0, 0)),
        ] + [wspec(a) for a in args] + [
            pl.BlockSpec(clf_w.shape, lambda g: (0, 0)),
            pl.BlockSpec((1, Np), lambda g: (0, 0)),
        ],
        out_specs=pl.BlockSpec((1, G, Np), lambda g: (g, 0, 0)),
        scratch_shapes=[pltpu.VMEM((p0 + M + p0, 48), jnp.bfloat16),
                        pltpu.VMEM((p0 + M + p0, 64), jnp.bfloat16)],
        compiler_params=pltpu.CompilerParams(
            dimension_semantics=("parallel",),
            vmem_limit_bytes=_VMEM_LIMIT,
        ),
    )(xf, mask, P, *args, clf_w, clf_b.reshape(1, Np))
    return out.reshape(B, Np)


_VC = 28


def _c1p_kernel(xr_ref, wb_ref, s_ref, b_ref, o_ref):
    acc = None
    for c in range(3):
        plane = xr_ref[0, c]
        he = plane[:, 0:224]
        ho = plane[:, 224:448]
        for di in range(3):
            a = (he[0:111], ho[0:111], he[1:112])[di]
            d = jnp.dot(a, wb_ref[0, c, di], preferred_element_type=jnp.float32)
            acc = d if acc is None else acc + d
    y = jnp.maximum(acc + b_ref[...], 0.0).astype(jnp.bfloat16)
    vc = _VC
    tap_a = y[:, 0:vc * 64]
    tap_b = y[:, 64:(vc + 1) * 64]
    tap_c = y[:, (vc + 1) * 64:(2 * vc + 1) * 64]
    yw = jnp.maximum(jnp.maximum(tap_a, tap_b), tap_c)
    p = None
    for di in range(3):
        t = jnp.dot(s_ref[di], yw, preferred_element_type=jnp.float32)
        p = t if p is None else jnp.maximum(p, t)
    p = p.astype(jnp.bfloat16)
    tile = pl.program_id(0)
    zb = jnp.zeros((57, 64), jnp.bfloat16)

    @pl.when(tile == 0)
    def _():
        o_ref[0, 0, :, 0:64] = zb
        o_ref[0, 0, :, 64:(vc + 1) * 64] = p

    @pl.when(tile == 1)
    def _():
        o_ref[0, 0, :, 0:27 * 64] = p[:, 0:27 * 64]
        o_ref[0, 0, :, 27 * 64:28 * 64] = zb
        o_ref[0, 0, :, 28 * 64:(vc + 1) * 64] = zb


def _conv1_pool(x, conv1_w, conv1_b):
    B = x.shape[0]
    vc = _VC
    lanes = (2 * vc + 1) * 64
    xr = x.reshape(B, 3, 112, 448).astype(jnp.bfloat16)

    w_r = conv1_w.reshape(3, 3, 3, 64)
    wt = jnp.transpose(w_r, (2, 0, 1, 3))
    ind = np.zeros((2, 3, 224, 2 * vc + 1), np.float32)
    for t in range(2):
        for jp in range(2 * vc + 1):
            if jp <= vc:
                j = 2 * (vc * t + jp)
            else:
                j = 2 * (vc * t + jp - vc - 1) + 1
            if j <= 110:
                for kj in range(3):
                    ind[t, kj, 2 * j + kj, jp] = 1.0
    wband = jnp.einsum("tkmj,cdko->tcdmjo", jnp.asarray(ind, jnp.bfloat16), wt)
    wband = wband.reshape(2, 3, 3, 224, lanes).astype(jnp.bfloat16)

    sel = np.zeros((3, 57, 111), np.float32)
    for di in range(3):
        for u in range(55):
            sel[di, u + 1, 2 * u + di] = 1.0
    sel = jnp.asarray(sel, jnp.bfloat16)
    btile = jnp.tile(conv1_b, 2 * vc + 1).reshape(1, lanes)

    out = pl.pallas_call(
        _c1p_kernel,
        out_shape=jax.ShapeDtypeStruct((2, B, 57, (vc + 1) * 64), jnp.bfloat16),
        grid=(2, B),
        in_specs=[
            pl.BlockSpec((1, 3, 112, 448), lambda t, g: (g, 0, 0, 0)),
            pl.BlockSpec((1, 3, 3, 224, lanes), lambda t, g: (t, 0, 0, 0, 0)),
            pl.BlockSpec((3, 57, 111), lambda t, g: (0, 0, 0)),
            pl.BlockSpec((1, lanes), lambda t, g: (0, 0)),
        ],
        out_specs=pl.BlockSpec((1, 1, 57, (vc + 1) * 64), lambda t, g: (t, g, 0, 0)),
        compiler_params=pltpu.CompilerParams(
            dimension_semantics=("parallel", "parallel"),
            vmem_limit_bytes=_VMEM_LIMIT,
        ),
    )(xr, wband, sel, btile)
    yc = jnp.concatenate([out[0], out[1][:, :, :28 * 64]], axis=-1)
    return yc.reshape(B, 3249, 64)


def _pool_halo_flat(xf, B, H, W):
    x = _unflat_interior(xf, B, H, W)
    p = jax.lax.reduce_window(x, jnp.bfloat16(0), jax.lax.max,
                              (1, 3, 3, 1), (1, 2, 2, 1),
                              ((0, 0), (2, 2), (2, 2), (0, 0)))
    Hn = (H - 3) // 2 + 2 + 1
    Mh = Hn * Hn
    Mp = _ru(Mh, 16)
    C = p.shape[-1]
    pf = p.reshape(B, Mh, C)
    pf = jnp.pad(pf, ((0, 0), (0, Mp - Mh), (0, 0)))
    return pf.reshape(B * Mp, C)


def _halo_flat(x):
    B, H, W, C = x.shape
    Mh = (H + 2) * (W + 2)
    Mp = _ru(Mh, 16)
    xp = jnp.pad(x, ((0, 0), (1, 1), (1, 1), (0, 0)))
    xf = xp.reshape(B, Mh, C)
    xf = jnp.pad(xf, ((0, 0), (0, Mp - Mh), (0, 0)))
    return xf.reshape(B * Mp, C)


def _unflat_interior(xf, B, H, W):
    Mh = (H + 2) * (W + 2)
    Mp = _ru(Mh, 16)
    C = xf.shape[1]
    x = xf.reshape(B, Mp, C)[:, :Mh].reshape(B, H + 2, W + 2, C)
    return x[:, 1:H + 1, 1:W + 1, :]


def _mask_flat(B, H, W):
    Mh = (H + 2) * (W + 2)
    Mp = _ru(Mh, 16)
    m = np.zeros((Mp, 1), np.float32)
    mm = np.zeros((H + 2, W + 2), np.float32)
    mm[1:H + 1, 1:W + 1] = 1.0
    m[:Mh, 0] = mm.reshape(-1)
    return jnp.asarray(np.tile(m, (B, 1)), jnp.bfloat16)


def kernel(x, conv1_w, conv1_b,
           f0_sq_w, f0_sq_b, f0_e1_w, f0_e1_b, f0_e3_w, f0_e3_b,
           f1_sq_w, f1_sq_b, f1_e1_w, f1_e1_b, f1_e3_w, f1_e3_b,
           f2_sq_w, f2_sq_b, f2_e1_w, f2_e1_b, f2_e3_w, f2_e3_b,
           f3_sq_w, f3_sq_b, f3_e1_w, f3_e1_b, f3_e3_w, f3_e3_b,
           f4_sq_w, f4_sq_b, f4_e1_w, f4_e1_b, f4_e3_w, f4_e3_b,
           f5_sq_w, f5_sq_b, f5_e1_w, f5_e1_b, f5_e3_w, f5_e3_b,
           f6_sq_w, f6_sq_b, f6_e1_w, f6_e1_b, f6_e3_w, f6_e3_b,
           f7_sq_w, f7_sq_b, f7_e1_w, f7_e1_b, f7_e3_w, f7_e3_b,
           clf_w, clf_b):
    fires = [
        {"sq_w": f0_sq_w, "sq_b": f0_sq_b, "e1_w": f0_e1_w, "e1_b": f0_e1_b, "e3_w": f0_e3_w, "e3_b": f0_e3_b},
        {"sq_w": f1_sq_w, "sq_b": f1_sq_b, "e1_w": f1_e1_w, "e1_b": f1_e1_b, "e3_w": f1_e3_w, "e3_b": f1_e3_b},
        {"sq_w": f2_sq_w, "sq_b": f2_sq_b, "e1_w": f2_e1_w, "e1_b": f2_e1_b, "e3_w": f2_e3_w, "e3_b": f2_e3_b},
        {"sq_w": f3_sq_w, "sq_b": f3_sq_b, "e1_w": f3_e1_w, "e1_b": f3_e1_b, "e3_w": f3_e3_w, "e3_b": f3_e3_b},
        {"sq_w": f4_sq_w, "sq_b": f4_sq_b, "e1_w": f4_e1_w, "e1_b": f4_e1_b, "e3_w": f4_e3_w, "e3_b": f4_e3_b},
        {"sq_w": f5_sq_w, "sq_b": f5_sq_b, "e1_w": f5_e1_w, "e1_b": f5_e1_b, "e3_w": f5_e3_w, "e3_b": f5_e3_b},
        {"sq_w": f6_sq_w, "sq_b": f6_sq_b, "e1_w": f6_e1_w, "e1_b": f6_e1_b, "e3_w": f6_e3_w, "e3_b": f6_e3_b},
        {"sq_w": f7_sq_w, "sq_b": f7_sq_b, "e1_w": f7_e1_w, "e1_b": f7_e1_b, "e3_w": f7_e3_w, "e3_b": f7_e3_b},
    ]
    B = x.shape[0]
    y = _conv1_pool(x, conv1_w, conv1_b)
    xf = jnp.pad(y, ((0, 0), (0, 3264 - 3249), (0, 0))).reshape(B * 3264, 64)

    m55 = _mask_flat(B, 55, 55)
    m27 = _mask_flat(B, 27, 27)
    m13 = _mask_flat(B, 13, 13)

    xf = _fire_pair(xf, m55, fires[0], fires[1], H=55, W=55, G=4)
    xf = _pool_halo_flat(xf, B, 55, 55)
    xf = _fire_pair(xf, m27, fires[2], fires[3], H=27, W=27, G=6)
    xf = _pool_halo_flat(xf, B, 27, 27)

    logits = _tail(xf, m13, fires[4:], clf_w, clf_b, H=13, W=13, G=6)
    return logits[:, :1000, None, None].astype(jnp.float32)


# speedup vs baseline: 1.0234x; 1.0234x over previous
import functools

import jax
import jax.numpy as jnp
import numpy as np
from jax.experimental import pallas as pl
from jax.experimental.pallas import tpu as pltpu

_VMEM_LIMIT = 32 * 1024 * 1024


def _ru(x, m):
    return (x + m - 1) // m * m


def _fire_apply(a, mask, wsq, bsq, w1, b1, w3, b3, s_scr, *, Wp, M, p0):
    sq = wsq.shape[1]
    s = jnp.dot(a, wsq, preferred_element_type=jnp.float32)
    s = jnp.maximum(s + bsq, 0.0)
    s = (s * mask).astype(jnp.bfloat16)
    s_scr[:p0] = jnp.zeros((p0, sq), jnp.bfloat16)
    s_scr[p0:p0 + M] = s
    s_scr[p0 + M:] = jnp.zeros((s_scr.shape[0] - p0 - M, sq), jnp.bfloat16)
    acc1 = jnp.dot(s, w1, preferred_element_type=jnp.float32) + b1
    acc3 = jnp.dot(s, w3[4 * sq:5 * sq, :], preferred_element_type=jnp.float32)
    for t in range(9):
        if t == 4:
            continue
        off = p0 + (t // 3 - 1) * Wp + (t % 3 - 1)
        acc3 = acc3 + jnp.dot(s_scr[off:off + M, :], w3[t * sq:(t + 1) * sq, :],
                              preferred_element_type=jnp.float32)
    acc3 = acc3 + b3
    return jnp.concatenate([jnp.maximum(acc1, 0.0).astype(jnp.bfloat16),
                            jnp.maximum(acc3, 0.0).astype(jnp.bfloat16)], axis=-1)


def _fire_pair_kernel(x_ref, mask_ref, wsq0, bsq0, w10, b10, w30, b30,
                      wsq1, bsq1, w11, b11, w31, b31, o_ref, s_scr,
                      *, Wp, M, p0):
    mask = mask_ref[...]
    y0 = _fire_apply(x_ref[...], mask, wsq0[...], bsq0[...], w10[...], b10[...],
                     w30[...], b30[...], s_scr, Wp=Wp, M=M, p0=p0)
    o_ref[...] = _fire_apply(y0, mask, wsq1[...], bsq1[...], w11[...], b11[...],
                             w31[...], b31[...], s_scr, Wp=Wp, M=M, p0=p0)


def _fire_pair(xf, mask, f0, f1, *, H, W, G):
    Wp = W + 2
    Mp = _ru((H + 2) * Wp, 16)
    R, Cin = xf.shape
    B = R // Mp
    M = G * Mp
    p0 = _ru(Wp + 1, 16)
    sq = f0["sq_w"].shape[1]
    E = f1["e1_w"].shape[1] + f1["e3_w"].shape[1]
    wspec = lambda a: pl.BlockSpec(a.shape, lambda g: (0, 0))
    return pl.pallas_call(
        functools.partial(_fire_pair_kernel, Wp=Wp, M=M, p0=p0),
        out_shape=jax.ShapeDtypeStruct((R, E), jnp.bfloat16),
        grid=(B // G,),
        in_specs=[
            pl.BlockSpec((M, Cin), lambda g: (g, 0)),
            pl.BlockSpec((M, 1), lambda g: (g, 0)),
            wspec(f0["sq_w"]), wspec(f0["sq_b"]), wspec(f0["e1_w"]),
            wspec(f0["e1_b"]), wspec(f0["e3_w"]), wspec(f0["e3_b"]),
            wspec(f1["sq_w"]), wspec(f1["sq_b"]), wspec(f1["e1_w"]),
            wspec(f1["e1_b"]), wspec(f1["e3_w"]), wspec(f1["e3_b"]),
        ],
        out_specs=pl.BlockSpec((M, E), lambda g: (g, 0)),
        scratch_shapes=[pltpu.VMEM((p0 + M + p0, sq), jnp.bfloat16)],
        compiler_params=pltpu.CompilerParams(
            dimension_semantics=("parallel",),
            vmem_limit_bytes=_VMEM_LIMIT,
        ),
    )(xf, mask,
      f0["sq_w"], f0["sq_b"], f0["e1_w"], f0["e1_b"], f0["e3_w"], f0["e3_b"],
      f1["sq_w"], f1["sq_b"], f1["e1_w"], f1["e1_b"], f1["e3_w"], f1["e3_b"])


def _tail_kernel(x_ref, mask_ref, p_ref,
                 wsq4, bsq4, w14, b14, w34, b34,
                 wsq5, bsq5, w15, b15, w35, b35,
                 wsq6, bsq6, w16, b16, w36, b36,
                 wsq7, bsq7, w17, b17, w37, b37,
                 cw_ref, cb_ref, o_ref, s48, s64, *, Wp, M, p0):
    mask = mask_ref[...]
    y = _fire_apply(x_ref[...], mask, wsq4[...], bsq4[...], w14[...], b14[...],
                    w34[...], b34[...], s48, Wp=Wp, M=M, p0=p0)
    y = _fire_apply(y, mask, wsq5[...], bsq5[...], w15[...], b15[...],
                    w35[...], b35[...], s48, Wp=Wp, M=M, p0=p0)
    y = _fire_apply(y, mask, wsq6[...], bsq6[...], w16[...], b16[...],
                    w36[...], b36[...], s64, Wp=Wp, M=M, p0=p0)
    y = _fire_apply(y, mask, wsq7[...], bsq7[...], w17[...], b17[...],
                    w37[...], b37[...], s64, Wp=Wp, M=M, p0=p0)
    acc = jnp.dot(y, cw_ref[...], preferred_element_type=jnp.float32)
    acc = jnp.maximum(acc + cb_ref[...], 0.0)
    o_ref[0] = jnp.dot(p_ref[...], acc, preferred_element_type=jnp.float32)


def _tail(xf, mask, fires, clf_w, clf_b, *, H, W, G):
    Wp = W + 2
    Mp = _ru((H + 2) * Wp, 16)
    R, Cin = xf.shape
    B = R // Mp
    M = G * Mp
    p0 = _ru(Wp + 1, 16)
    Np = clf_w.shape[1]
    mvec = np.zeros((Mp,), np.float32)
    mm = np.zeros((H + 2, W + 2), np.float32)
    mm[1:H + 1, 1:W + 1] = 1.0 / float(H * W)
    mvec[:(H + 2) * Wp] = mm.reshape(-1)
    P = np.zeros((G, M), np.float32)
    for g in range(G):
        P[g, g * Mp:(g + 1) * Mp] = mvec
    P = jnp.asarray(P)
    wspec = lambda a: pl.BlockSpec(a.shape, lambda g: (0, 0))
    args = []
    for f in fires:
        args += [f["sq_w"], f["sq_b"], f["e1_w"], f["e1_b"], f["e3_w"], f["e3_b"]]
    out = pl.pallas_call(
        functools.partial(_tail_kernel, Wp=Wp, M=M, p0=p0),
        out_shape=jax.ShapeDtypeStruct((B // G, G, Np), jnp.float32),
        grid=(B // G,),
        in_specs=[
            pl.BlockSpec((M, Cin), lambda g: (g, 0)),
            pl.BlockSpec((M, 1), lambda g: (g, 0)),
            pl.BlockSpec((G, M), lambda g: (0, 0)),
        ] + [wspec(a) for a in args] + [
            pl.BlockSpec(clf_w.shape, lambda g: (0, 0)),
            pl.BlockSpec((1, Np), lambda g: (0, 0)),
        ],
        out_specs=pl.BlockSpec((1, G, Np), lambda g: (g, 0, 0)),
        scratch_shapes=[pltpu.VMEM((p0 + M + p0, 48), jnp.bfloat16),
                        pltpu.VMEM((p0 + M + p0, 64), jnp.bfloat16)],
        compiler_params=pltpu.CompilerParams(
            dimension_semantics=("parallel",),
            vmem_limit_bytes=_VMEM_LIMIT,
        ),
    )(xf, mask, P, *args, clf_w, clf_b.reshape(1, Np))
    return out.reshape(B, Np)


_VC = 28


def _c1p_kernel(xr_ref, wb_ref, s_ref, b_ref, o_ref):
    acc = None
    for c in range(3):
        plane = xr_ref[0, c]
        he = plane[:, 0:224]
        ho = plane[:, 224:448]
        for di in range(3):
            a = (he[0:111], ho[0:111], he[1:112])[di]
            d = jnp.dot(a, wb_ref[0, c, di], preferred_element_type=jnp.float32)
            acc = d if acc is None else acc + d
    y = jnp.maximum(acc + b_ref[...], 0.0).astype(jnp.bfloat16)
    vc = _VC
    tap_a = y[:, 0:vc * 64]
    tap_b = y[:, 64:(vc + 1) * 64]
    tap_c = y[:, (vc + 1) * 64:(2 * vc + 1) * 64]
    yw = jnp.maximum(jnp.maximum(tap_a, tap_b), tap_c)
    p = None
    for di in range(3):
        t = jnp.dot(s_ref[di], yw, preferred_element_type=jnp.float32)
        p = t if p is None else jnp.maximum(p, t)
    p = p.astype(jnp.bfloat16)
    tile = pl.program_id(0)
    zb = jnp.zeros((57, 64), jnp.bfloat16)

    @pl.when(tile == 0)
    def _():
        o_ref[0, 0, :, 0:64] = zb
        o_ref[0, 0, :, 64:(vc + 1) * 64] = p

    @pl.when(tile == 1)
    def _():
        o_ref[0, 0, :, 0:27 * 64] = p[:, 0:27 * 64]
        o_ref[0, 0, :, 27 * 64:28 * 64] = zb
        o_ref[0, 0, :, 28 * 64:(vc + 1) * 64] = zb


def _conv1_pool(x, conv1_w, conv1_b):
    B = x.shape[0]
    vc = _VC
    lanes = (2 * vc + 1) * 64
    xr = x.reshape(B, 3, 112, 448).astype(jnp.bfloat16)

    w_r = conv1_w.reshape(3, 3, 3, 64)
    wt = jnp.transpose(w_r, (2, 0, 1, 3))
    ind = np.zeros((2, 3, 224, 2 * vc + 1), np.float32)
    for t in range(2):
        for jp in range(2 * vc + 1):
            if jp <= vc:
                j = 2 * (vc * t + jp)
            else:
                j = 2 * (vc * t + jp - vc - 1) + 1
            if j <= 110:
                for kj in range(3):
                    ind[t, kj, 2 * j + kj, jp] = 1.0
    wband = jnp.einsum("tkmj,cdko->tcdmjo", jnp.asarray(ind, jnp.bfloat16), wt)
    wband = wband.reshape(2, 3, 3, 224, lanes).astype(jnp.bfloat16)

    sel = np.zeros((3, 57, 111), np.float32)
    for di in range(3):
        for u in range(55):
            sel[di, u + 1, 2 * u + di] = 1.0
    sel = jnp.asarray(sel, jnp.bfloat16)
    btile = jnp.tile(conv1_b, 2 * vc + 1).reshape(1, lanes)

    out = pl.pallas_call(
        _c1p_kernel,
        out_shape=jax.ShapeDtypeStruct((2, B, 57, (vc + 1) * 64), jnp.bfloat16),
        grid=(2, B),
        in_specs=[
            pl.BlockSpec((1, 3, 112, 448), lambda t, g: (g, 0, 0, 0)),
            pl.BlockSpec((1, 3, 3, 224, lanes), lambda t, g: (t, 0, 0, 0, 0)),
            pl.BlockSpec((3, 57, 111), lambda t, g: (0, 0, 0)),
            pl.BlockSpec((1, lanes), lambda t, g: (0, 0)),
        ],
        out_specs=pl.BlockSpec((1, 1, 57, (vc + 1) * 64), lambda t, g: (t, g, 0, 0)),
        compiler_params=pltpu.CompilerParams(
            dimension_semantics=("parallel", "parallel"),
            vmem_limit_bytes=_VMEM_LIMIT,
        ),
    )(xr, wband, sel, btile)
    yc = jnp.concatenate([out[0], out[1][:, :, :28 * 64]], axis=-1)
    return yc.reshape(B, 3249, 64)


def _pool_halo_flat(xf, B, H, W):
    x = _unflat_interior(xf, B, H, W)
    p = jax.lax.reduce_window(x, jnp.bfloat16(0), jax.lax.max,
                              (1, 3, 3, 1), (1, 2, 2, 1),
                              ((0, 0), (2, 2), (2, 2), (0, 0)))
    Hn = (H - 3) // 2 + 2 + 1
    Mh = Hn * Hn
    Mp = _ru(Mh, 16)
    C = p.shape[-1]
    pf = p.reshape(B, Mh, C)
    pf = jnp.pad(pf, ((0, 0), (0, Mp - Mh), (0, 0)))
    return pf.reshape(B * Mp, C)


def _halo_flat(x):
    B, H, W, C = x.shape
    Mh = (H + 2) * (W + 2)
    Mp = _ru(Mh, 16)
    xp = jnp.pad(x, ((0, 0), (1, 1), (1, 1), (0, 0)))
    xf = xp.reshape(B, Mh, C)
    xf = jnp.pad(xf, ((0, 0), (0, Mp - Mh), (0, 0)))
    return xf.reshape(B * Mp, C)


def _unflat_interior(xf, B, H, W):
    Mh = (H + 2) * (W + 2)
    Mp = _ru(Mh, 16)
    C = xf.shape[1]
    x = xf.reshape(B, Mp, C)[:, :Mh].reshape(B, H + 2, W + 2, C)
    return x[:, 1:H + 1, 1:W + 1, :]


def _mask_flat(B, H, W):
    Mh = (H + 2) * (W + 2)
    Mp = _ru(Mh, 16)
    m = np.zeros((Mp, 1), np.float32)
    mm = np.zeros((H + 2, W + 2), np.float32)
    mm[1:H + 1, 1:W + 1] = 1.0
    m[:Mh, 0] = mm.reshape(-1)
    return jnp.asarray(np.tile(m, (B, 1)), jnp.bfloat16)


def kernel(x, conv1_w, conv1_b,
           f0_sq_w, f0_sq_b, f0_e1_w, f0_e1_b, f0_e3_w, f0_e3_b,
           f1_sq_w, f1_sq_b, f1_e1_w, f1_e1_b, f1_e3_w, f1_e3_b,
           f2_sq_w, f2_sq_b, f2_e1_w, f2_e1_b, f2_e3_w, f2_e3_b,
           f3_sq_w, f3_sq_b, f3_e1_w, f3_e1_b, f3_e3_w, f3_e3_b,
           f4_sq_w, f4_sq_b, f4_e1_w, f4_e1_b, f4_e3_w, f4_e3_b,
           f5_sq_w, f5_sq_b, f5_e1_w, f5_e1_b, f5_e3_w, f5_e3_b,
           f6_sq_w, f6_sq_b, f6_e1_w, f6_e1_b, f6_e3_w, f6_e3_b,
           f7_sq_w, f7_sq_b, f7_e1_w, f7_e1_b, f7_e3_w, f7_e3_b,
           clf_w, clf_b):
    fires = [
        {"sq_w": f0_sq_w, "sq_b": f0_sq_b, "e1_w": f0_e1_w, "e1_b": f0_e1_b, "e3_w": f0_e3_w, "e3_b": f0_e3_b},
        {"sq_w": f1_sq_w, "sq_b": f1_sq_b, "e1_w": f1_e1_w, "e1_b": f1_e1_b, "e3_w": f1_e3_w, "e3_b": f1_e3_b},
        {"sq_w": f2_sq_w, "sq_b": f2_sq_b, "e1_w": f2_e1_w, "e1_b": f2_e1_b, "e3_w": f2_e3_w, "e3_b": f2_e3_b},
        {"sq_w": f3_sq_w, "sq_b": f3_sq_b, "e1_w": f3_e1_w, "e1_b": f3_e1_b, "e3_w": f3_e3_w, "e3_b": f3_e3_b},
        {"sq_w": f4_sq_w, "sq_b": f4_sq_b, "e1_w": f4_e1_w, "e1_b": f4_e1_b, "e3_w": f4_e3_w, "e3_b": f4_e3_b},
        {"sq_w": f5_sq_w, "sq_b": f5_sq_b, "e1_w": f5_e1_w, "e1_b": f5_e1_b, "e3_w": f5_e3_w, "e3_b": f5_e3_b},
        {"sq_w": f6_sq_w, "sq_b": f6_sq_b, "e1_w": f6_e1_w, "e1_b": f6_e1_b, "e3_w": f6_e3_w, "e3_b": f6_e3_b},
        {"sq_w": f7_sq_w, "sq_b": f7_sq_b, "e1_w": f7_e1_w, "e1_b": f7_e1_b, "e3_w": f7_e3_w, "e3_b": f7_e3_b},
    ]
    B = x.shape[0]
    y = _conv1_pool(x, conv1_w, conv1_b)
    xf = jnp.pad(y, ((0, 0), (0, 3264 - 3249), (0, 0))).reshape(B * 3264, 64)

    m55 = _mask_flat(B, 55, 55)
    m27 = _mask_flat(B, 27, 27)
    m13 = _mask_flat(B, 13, 13)

    xf = _fire_pair(xf, m55, fires[0], fires[1], H=55, W=55, G=2)
    xf = _pool_halo_flat(xf, B, 55, 55)
    xf = _fire_pair(xf, m27, fires[2], fires[3], H=27, W=27, G=6)
    xf = _pool_halo_flat(xf, B, 27, 27)

    logits = _tail(xf, m13, fires[4:], clf_w, clf_b, H=13, W=13, G=6)
    return logits[:, :1000, None, None].astype(jnp.float32)
